# Initial kernel scaffold; baseline (speedup 1.0000x reference)
import jax
import jax.numpy as jnp
from jax.experimental import pallas as pl


def kernel(pts, codebook_0, codebook_1, codebook_2, codebook_3, codebook_4):
    raise NotImplementedError("write your pallas kernel here")


# speedup vs baseline: 7.0241x; 7.0241x over previous
import functools

import jax
import jax.numpy as jnp
from jax import lax
from jax.experimental import pallas as pl
from jax.experimental.pallas import tpu as pltpu
from jax.experimental.pallas import tpu_sc as plsc

_BASE = 4
_NLOD = 5
_FEAT = 4
_RES = [2 ** (l + _BASE) for l in range(_NLOD)]

_L = 16
_B = 512
_ISUB = 128


def _lerp(a, b, w):
    return a + w * (b - a)


def _sc_body(x_hbm, y_hbm, z_hbm, cb0, cb1, cb2, cb3, cb4, out_hbm,
             x_v, y_v, z_v, idx_v, w_v, par_v, rows_v, acc_v, sem,
             *, n_pts, nw):
    cbs = [cb0, cb1, cb2, cb3, cb4]
    ppw = n_pts // nw
    nchunk = ppw // _B

    wid = lax.axis_index("s") * 2 + lax.axis_index("c")
    base = wid * ppw

    iota = lax.iota(jnp.int32, _L)
    rep4 = lax.shift_right_logical(iota, 2)
    feat4 = lax.bitwise_and(iota, 3)

    def compute_idx(res):
        scale = jnp.float32(0.5 * (res - 1))
        hi = jnp.float32(res - 1 - 1e-05)
        res2 = res * res
        poffs = [0, res // 2, res2 // 2, (res2 + res) // 2]

        def body(j, _):
            sl = pl.ds(j * _L, _L)
            xs = x_v[sl] * scale + scale
            ys = y_v[sl] * scale + scale
            zs = z_v[sl] * scale + scale
            cx = jnp.minimum(jnp.maximum(xs, 0.0), hi).astype(jnp.int32)
            cy = jnp.minimum(jnp.maximum(ys, 0.0), hi).astype(jnp.int32)
            cz = jnp.minimum(jnp.maximum(zs, 0.0), hi).astype(jnp.int32)
            ib = (cz * res + cy) * res + cx
            par = lax.bitwise_and(ib, 1)
            ibh = lax.shift_right_logical(ib, 1)
            for c in range(4):
                r0 = ibh + poffs[c]
                idx_v[pl.ds((2 * c) * _B + j * _L, _L)] = r0
                idx_v[pl.ds((2 * c + 1) * _B + j * _L, _L)] = r0 + par
            par_v[sl] = par
            w_v[pl.ds(j * _L, _L)] = xs - cx.astype(jnp.float32)
            w_v[pl.ds(_B + j * _L, _L)] = ys - cy.astype(jnp.float32)
            w_v[pl.ds(2 * _B + j * _L, _L)] = zs - cz.astype(jnp.float32)
            return 0

        lax.fori_loop(0, _B // _L, body, 0)

    def gather(cb):
        def issue(jj, _):
            for c in range(8):
                sl = pl.ds(c * _B + jj * _ISUB, _ISUB)
                pltpu.async_copy(cb.at[idx_v.at[sl]], rows_v.at[sl], sem)
            return 0

        def drain(jj, _):
            for c in range(8):
                sl = pl.ds(c * _B + jj * _ISUB, _ISUB)
                pltpu.make_async_copy(cb.at[idx_v.at[sl]],
                                      rows_v.at[sl], sem).wait()
            return 0

        lax.fori_loop(0, _B // _ISUB, issue, 0)
        lax.fori_loop(0, _B // _ISUB, drain, 0)

    def combine(first):
        def body(g, _):
            pid = g * 4 + rep4
            wxr = plsc.load_gather(w_v, [pid])
            wyr = plsc.load_gather(w_v, [pid + _B])
            wzr = plsc.load_gather(w_v, [pid + 2 * _B])
            par = plsc.load_gather(par_v, [pid])
            col1 = lax.shift_left(par, 2) + feat4
            col2 = 4 - lax.shift_left(par, 2) + feat4
            r = []
            for c in range(4):
                r.append(plsc.load_gather(
                    rows_v, [pid + (2 * c) * _B, col1]))
                r.append(plsc.load_gather(
                    rows_v, [pid + (2 * c + 1) * _B, col2]))
            t00 = _lerp(r[0], r[1], wxr)
            t01 = _lerp(r[2], r[3], wxr)
            t10 = _lerp(r[4], r[5], wxr)
            t11 = _lerp(r[6], r[7], wxr)
            u0 = _lerp(t00, t01, wyr)
            u1 = _lerp(t10, t11, wyr)
            v = jnp.maximum(_lerp(u0, u1, wzr), 0.0)
            sl = pl.ds(g * _L, _L)
            if first:
                acc_v[sl] = v
            else:
                acc_v[sl] = acc_v[sl] + v
            return 0

        lax.fori_loop(0, _B * _FEAT // _L, body, 0)

    def chunk(ch, _):
        pbase = base + ch * _B
        pltpu.sync_copy(x_hbm.at[pl.ds(pbase, _B)], x_v)
        pltpu.sync_copy(y_hbm.at[pl.ds(pbase, _B)], y_v)
        pltpu.sync_copy(z_hbm.at[pl.ds(pbase, _B)], z_v)
        for li in range(_NLOD):
            compute_idx(_RES[li])
            gather(cbs[li])
            combine(first=(li == 0))
        pltpu.sync_copy(acc_v, out_hbm.at[pl.ds(pbase * _FEAT, _B * _FEAT)])
        return 0

    lax.fori_loop(0, nchunk, chunk, 0)


def kernel(pts, codebook_0, codebook_1, codebook_2, codebook_3, codebook_4):
    n_pts = pts.shape[0]
    info = plsc.get_sparse_core_info()
    nw = info.num_cores * info.num_subcores
    assert n_pts % (nw * _B) == 0

    x = pts[:, 0]
    y = pts[:, 1]
    z = pts[:, 2]
    cbs = [cb.reshape(cb.shape[0] // 2, 2 * _FEAT)
           for cb in (codebook_0, codebook_1, codebook_2,
                      codebook_3, codebook_4)]

    mesh = plsc.VectorSubcoreMesh(core_axis_name="c", subcore_axis_name="s")
    body = functools.partial(_sc_body, n_pts=n_pts, nw=nw)
    run = pl.kernel(
        body,
        mesh=mesh,
        compiler_params=pltpu.CompilerParams(
            use_tc_tiling_on_sc=False, needs_layout_passes=False),
        out_type=jax.ShapeDtypeStruct((n_pts * _FEAT,), jnp.float32),
        scratch_types=[
            pltpu.VMEM((_B,), jnp.float32),
            pltpu.VMEM((_B,), jnp.float32),
            pltpu.VMEM((_B,), jnp.float32),
            pltpu.VMEM((8 * _B,), jnp.int32),
            pltpu.VMEM((3 * _B,), jnp.float32),
            pltpu.VMEM((_B,), jnp.int32),
            pltpu.VMEM((8 * _B, 2 * _FEAT), jnp.float32),
            pltpu.VMEM((_B * _FEAT,), jnp.float32),
            pltpu.SemaphoreType.DMA,
        ],
    )
    out = run(x, y, z, *cbs)
    return out.reshape(n_pts, _FEAT)


# speedup vs baseline: 42.5901x; 6.0634x over previous
import functools

import jax
import jax.numpy as jnp
from jax import lax
from jax.experimental import pallas as pl
from jax.experimental.pallas import tpu as pltpu
from jax.experimental.pallas import tpu_sc as plsc

_BASE = 4
_NLOD = 5
_FEAT = 4
_RES = [2 ** (l + _BASE) for l in range(_NLOD)]

_L = 16
_B = 512
_ISUB = 128
_PC = 2048


def _lerp(a, b, w):
    return a + w * (b - a)


def _sc_body(x_hbm, y_hbm, z_hbm, cb0, cb1, cb2, cb3, cb4, out_hbm,
             x_v, y_v, z_v, idx_v, w_v, par_v, rows_v, acc_v, sem0, sem1,
             *, n_pts, nw):
    cbs = [cb0, cb1, cb2, cb3, cb4]
    sems = [sem0, sem1]
    ppw = n_pts // nw
    nchunk = ppw // _B

    wid = lax.axis_index("s") * 2 + lax.axis_index("c")
    base = wid * ppw

    iota = lax.iota(jnp.int32, _L)
    rep4 = lax.shift_right_logical(iota, 2)
    feat4 = lax.bitwise_and(iota, 3)

    def compute_idx(res, pb):
        scale = jnp.float32(0.5 * (res - 1))
        hi = jnp.float32(res - 1 - 1e-05)
        res2 = res * res
        poffs = [0, res // 2, res2 // 2, (res2 + res) // 2]
        io = pb * 8 * _B
        wo = pb * 3 * _B

        def body(j, _):
            sl = pl.ds(j * _L, _L)
            xs = x_v[sl] * scale + scale
            ys = y_v[sl] * scale + scale
            zs = z_v[sl] * scale + scale
            cx = jnp.minimum(jnp.maximum(xs, 0.0), hi).astype(jnp.int32)
            cy = jnp.minimum(jnp.maximum(ys, 0.0), hi).astype(jnp.int32)
            cz = jnp.minimum(jnp.maximum(zs, 0.0), hi).astype(jnp.int32)
            ib = (cz * res + cy) * res + cx
            par = lax.bitwise_and(ib, 1)
            ibh = lax.shift_right_logical(ib, 1)
            for c in range(4):
                r0 = ibh + poffs[c]
                idx_v[pl.ds(io + (2 * c) * _B + j * _L, _L)] = r0
                idx_v[pl.ds(io + (2 * c + 1) * _B + j * _L, _L)] = r0 + par
            par_v[pl.ds(pb * _B + j * _L, _L)] = par
            w_v[pl.ds(wo + j * _L, _L)] = xs - cx.astype(jnp.float32)
            w_v[pl.ds(wo + _B + j * _L, _L)] = ys - cy.astype(jnp.float32)
            w_v[pl.ds(wo + 2 * _B + j * _L, _L)] = zs - cz.astype(jnp.float32)
            return 0

        lax.fori_loop(0, _B // _L, body, 0)

    def gather_issue(cb, pb):
        def issue(jj, _):
            for c in range(8):
                o = c * _B + jj * _ISUB
                pltpu.async_copy(
                    cb.at[idx_v.at[pl.ds(pb * 8 * _B + o, _ISUB)]],
                    rows_v.at[pl.ds(pb * 8 * _B + o, _ISUB)], sems[pb])
            return 0

        lax.fori_loop(0, _B // _ISUB, issue, 0)

    def gather_wait(cb, pb):
        def drain(jj, _):
            for c in range(8):
                o = c * _B + jj * _ISUB
                pltpu.make_async_copy(
                    cb.at[idx_v.at[pl.ds(pb * 8 * _B + o, _ISUB)]],
                    rows_v.at[pl.ds(pb * 8 * _B + o, _ISUB)],
                    sems[pb]).wait()
            return 0

        lax.fori_loop(0, _B // _ISUB, drain, 0)

    def combine(first, pb):
        ro = pb * 8 * _B
        wo = pb * 3 * _B

        def body(g, _):
            pid = g * 4 + rep4
            wxr = plsc.load_gather(w_v, [pid + wo])
            wyr = plsc.load_gather(w_v, [pid + wo + _B])
            wzr = plsc.load_gather(w_v, [pid + wo + 2 * _B])
            par = plsc.load_gather(par_v, [pid + pb * _B])
            col1 = lax.shift_left(par, 2) + feat4
            col2 = 4 - lax.shift_left(par, 2) + feat4
            r = []
            for c in range(4):
                r.append(plsc.load_gather(
                    rows_v, [pid + ro + (2 * c) * _B, col1]))
                r.append(plsc.load_gather(
                    rows_v, [pid + ro + (2 * c + 1) * _B, col2]))
            t00 = _lerp(r[0], r[1], wxr)
            t01 = _lerp(r[2], r[3], wxr)
            t10 = _lerp(r[4], r[5], wxr)
            t11 = _lerp(r[6], r[7], wxr)
            u0 = _lerp(t00, t01, wyr)
            u1 = _lerp(t10, t11, wyr)
            v = jnp.maximum(_lerp(u0, u1, wzr), 0.0)
            sl = pl.ds(g * _L, _L)
            if first:
                acc_v[sl] = v
            else:
                acc_v[sl] = acc_v[sl] + v
            return 0

        lax.fori_loop(0, _B * _FEAT // _L, body, 0)

    def chunk(ch, _):
        pbase = base + ch * _B
        pltpu.sync_copy(x_hbm.at[pl.ds(pbase, _B)], x_v)
        pltpu.sync_copy(y_hbm.at[pl.ds(pbase, _B)], y_v)
        pltpu.sync_copy(z_hbm.at[pl.ds(pbase, _B)], z_v)
        compute_idx(_RES[0], 0)
        gather_issue(cbs[0], 0)
        for li in range(_NLOD):
            pb = li & 1
            if li + 1 < _NLOD:
                compute_idx(_RES[li + 1], 1 - pb)
                gather_issue(cbs[li + 1], 1 - pb)
            gather_wait(cbs[li], pb)
            combine(li == 0, pb)
        pltpu.sync_copy(acc_v, out_hbm.at[pl.ds(pbase * _FEAT, _B * _FEAT)])
        return 0

    lax.fori_loop(0, nchunk, chunk, 0)


def _pre_body(*refs, vols):
    planes, outs = refs[:20], refs[20:25]
    pb_v, ob_v, semi, semo = refs[25:]
    wid = lax.axis_index("s") * 2 + lax.axis_index("c")

    iota = lax.iota(jnp.int32, _L)
    feat = lax.bitwise_and(iota, 3)
    ebase = (lax.shift_left(lax.shift_right_logical(iota, 3), 1)
             + lax.bitwise_and(lax.shift_right_logical(iota, 2), 1))

    for l, v in enumerate(vols):
        rows_pw = v // 2 // 32
        c = min(_PC, rows_pw)
        iters = rows_pw // c
        out = outs[l]
        pls = planes[4 * l: 4 * l + 4]

        def it(t, _, *, rows_pw=rows_pw, c=c, out=out, pls=pls):
            r0 = wid * rows_pw + t * c
            cps = [pltpu.async_copy(p.at[pl.ds(r0 * 2, c * 2)],
                                    pb_v.at[f, pl.ds(0, c * 2)], semi)
                   for f, p in enumerate(pls)]
            for cp in cps:
                cp.wait()

            @pl.when(t > 0)
            def _():
                pltpu.make_async_copy(
                    ob_v.at[pl.ds(0, c * 8)],
                    out.at[pl.ds(r0 * 8, c * 8)], semo).wait()

            def vg(m, _2):
                for u in range(4):
                    ev = (m * 4 + u) * 4 + ebase
                    ob_v[pl.ds((m * 4 + u) * _L, _L)] = (
                        plsc.load_gather(pb_v, [feat, ev]))
                return 0

            lax.fori_loop(0, c * 8 // _L // 4, vg, 0)
            pltpu.async_copy(ob_v.at[pl.ds(0, c * 8)],
                             out.at[pl.ds(r0 * 8, c * 8)], semo)
            return 0

        lax.fori_loop(0, iters, it, 0)
        pltpu.make_async_copy(
            ob_v.at[pl.ds(0, c * 8)],
            out.at[pl.ds(wid * rows_pw * 8, c * 8)], semo).wait()


def kernel(pts, codebook_0, codebook_1, codebook_2, codebook_3, codebook_4):
    n_pts = pts.shape[0]
    info = plsc.get_sparse_core_info()
    nw = info.num_cores * info.num_subcores
    assert n_pts % (nw * _B) == 0

    x = pts[:, 0]
    y = pts[:, 1]
    z = pts[:, 2]

    raw = (codebook_0, codebook_1, codebook_2, codebook_3, codebook_4)
    vols = tuple(cb.shape[0] for cb in raw)
    planes = [cb[:, f] for cb in raw for f in range(4)]

    mesh = plsc.VectorSubcoreMesh(core_axis_name="c", subcore_axis_name="s")
    pre = pl.kernel(
        functools.partial(_pre_body, vols=vols),
        mesh=mesh,
        compiler_params=pltpu.CompilerParams(
            use_tc_tiling_on_sc=False, needs_layout_passes=False),
        out_type=tuple(jax.ShapeDtypeStruct((v * 4,), jnp.float32)
                       for v in vols),
        scratch_types=[
            pltpu.VMEM((4, 2 * _PC), jnp.float32),
            pltpu.VMEM((8 * _PC,), jnp.float32),
            pltpu.SemaphoreType.DMA,
            pltpu.SemaphoreType.DMA,
        ],
    )
    flats = pre(*planes)
    cbs = [f.reshape(v // 2, 2 * _FEAT) for f, v in zip(flats, vols)]

    body = functools.partial(_sc_body, n_pts=n_pts, nw=nw)
    run = pl.kernel(
        body,
        mesh=mesh,
        compiler_params=pltpu.CompilerParams(
            use_tc_tiling_on_sc=False, needs_layout_passes=False),
        out_type=jax.ShapeDtypeStruct((n_pts * _FEAT,), jnp.float32),
        scratch_types=[
            pltpu.VMEM((_B,), jnp.float32),
            pltpu.VMEM((_B,), jnp.float32),
            pltpu.VMEM((_B,), jnp.float32),
            pltpu.VMEM((2 * 8 * _B,), jnp.int32),
            pltpu.VMEM((2 * 3 * _B,), jnp.float32),
            pltpu.VMEM((2 * _B,), jnp.int32),
            pltpu.VMEM((2 * 8 * _B, 2 * _FEAT), jnp.float32),
            pltpu.VMEM((_B * _FEAT,), jnp.float32),
            pltpu.SemaphoreType.DMA,
            pltpu.SemaphoreType.DMA,
        ],
    )
    out = run(x, y, z, *cbs)
    return out.reshape(n_pts, _FEAT)
